# Initial kernel scaffold; baseline (speedup 1.0000x reference)
import jax
import jax.numpy as jnp
from jax.experimental import pallas as pl


def kernel(x, real_embed, imag_embed):
    raise NotImplementedError("write your pallas kernel here")


# speedup vs baseline: 1.0452x; 1.0452x over previous
import functools

import jax
import jax.numpy as jnp
from jax import lax
from jax.experimental import pallas as pl
from jax.experimental.pallas import tpu as pltpu
from jax.experimental.pallas import tpu_sc as plsc

NC = 2
NS = 16
NW = NC * NS
CH = 128
FEAT = 32


@functools.lru_cache(maxsize=None)
def _build(n_blocks, feat):
    nch = n_blocks // NW
    b_flat = n_blocks * CH
    mesh = plsc.VectorSubcoreMesh(
        core_axis_name="c", subcore_axis_name="s", num_cores=NC, num_subcores=NS
    )
    out_sds = jax.ShapeDtypeStruct((b_flat, feat), jnp.bfloat16)

    @functools.partial(
        pl.kernel,
        out_type=(out_sds, out_sds),
        mesh=mesh,
        compiler_params=pltpu.CompilerParams(
            needs_layout_passes=False, use_tc_tiling_on_sc=False
        ),
        scratch_types=[
            pltpu.VMEM((nch, CH), jnp.int32),
            pltpu.VMEM((CH, feat), jnp.float32),
            pltpu.VMEM((CH, feat), jnp.float32),
            pltpu.VMEM((CH, feat), jnp.bfloat16),
            pltpu.VMEM((CH, feat), jnp.bfloat16),
            pltpu.SemaphoreType.DMA,
            pltpu.SemaphoreType.DMA,
        ],
    )
    def grab(x_hbm, real_hbm, imag_hbm, out_r_hbm, out_i_hbm,
             idx_v, buf_r, buf_i, ob_r, ob_i, sem_r, sem_i):
        wid = lax.axis_index("s") * NC + lax.axis_index("c")
        rbase = wid * nch
        pltpu.sync_copy(x_hbm.at[pl.ds(rbase, nch)], idx_v)

        lanes = lax.iota(jnp.int32, 16)
        idx_e = (lanes * 2) & 15
        idx_o = idx_e + 1
        lo_half = lanes < 8

        @pl.loop(0, nch)
        def _chunk(j):
            cp_r = pltpu.async_copy(real_hbm.at[idx_v.at[j]], buf_r, sem_r)
            cp_i = pltpu.async_copy(imag_hbm.at[idx_v.at[j]], buf_i, sem_i)
            cp_r.wait()
            cp_i.wait()

            @pl.loop(0, CH)
            def _row(i):
                for buf, ob in ((buf_r, ob_r), (buf_i, ob_i)):
                    a = buf[i, pl.ds(0, 16)]
                    b = buf[i, pl.ds(16, 16)]
                    e = jnp.where(lo_half, a[idx_e], b[idx_e])
                    o = jnp.where(lo_half, a[idx_o], b[idx_o])
                    ob[i, :] = plsc.pack(
                        e, o, format=plsc.PackFormat.INTERLEAVED
                    )

            off = (rbase + j) * CH
            pltpu.sync_copy(ob_r, out_r_hbm.at[pl.ds(off, CH)])
            pltpu.sync_copy(ob_i, out_i_hbm.at[pl.ds(off, CH)])

    return grab


def kernel(x, real_embed, imag_embed):
    batch, hist = x.shape
    feat = real_embed.shape[1]
    b_flat = batch * hist
    x2 = x.reshape(b_flat // CH, CH)
    fn = _build(b_flat // CH, feat)
    out_r, out_i = fn(x2, real_embed, imag_embed)
    return (out_r.reshape(batch, hist, feat),
            out_i.reshape(batch, hist, feat))


# speedup vs baseline: 1.0570x; 1.0114x over previous
import functools

import jax
import jax.numpy as jnp
from jax import lax
from jax.experimental import pallas as pl
from jax.experimental.pallas import tpu as pltpu
from jax.experimental.pallas import tpu_sc as plsc

NC = 2
NS = 16
NW = NC * NS
CH = 128


@functools.lru_cache(maxsize=None)
def _build(n_blocks, feat):
    nch = n_blocks // NW
    b_flat = n_blocks * CH
    mesh = plsc.VectorSubcoreMesh(
        core_axis_name="c", subcore_axis_name="s", num_cores=NC, num_subcores=NS
    )
    out_sds = jax.ShapeDtypeStruct((b_flat, feat), jnp.bfloat16)
    buf_t = pltpu.VMEM((CH, feat), jnp.bfloat16)

    @functools.partial(
        pl.kernel,
        out_type=(out_sds, out_sds),
        mesh=mesh,
        compiler_params=pltpu.CompilerParams(
            needs_layout_passes=False, use_tc_tiling_on_sc=False
        ),
        scratch_types=[
            pltpu.VMEM((nch, CH), jnp.int32),
            (buf_t, buf_t),
            (buf_t, buf_t),
            (pltpu.SemaphoreType.DMA, pltpu.SemaphoreType.DMA),
            (pltpu.SemaphoreType.DMA, pltpu.SemaphoreType.DMA),
        ],
    )
    def grab(x_hbm, real_hbm, imag_hbm, out_r_hbm, out_i_hbm,
             idx_v, buf_r, buf_i, sem_r, sem_i):
        wid = lax.axis_index("s") * NC + lax.axis_index("c")
        rbase = wid * nch
        pltpu.sync_copy(x_hbm.at[pl.ds(rbase, nch)], idx_v)

        def start(j, p):
            pltpu.async_copy(real_hbm.at[idx_v.at[j]], buf_r[p], sem_r[p])
            pltpu.async_copy(imag_hbm.at[idx_v.at[j]], buf_i[p], sem_i[p])

        def wait(j, p):
            pltpu.make_async_copy(
                real_hbm.at[idx_v.at[j]], buf_r[p], sem_r[p]).wait()
            pltpu.make_async_copy(
                imag_hbm.at[idx_v.at[j]], buf_i[p], sem_i[p]).wait()

        start(0, 0)
        start(1, 1)

        @pl.loop(0, nch, step=2)
        def _chunk(j):
            for p in range(2):
                jj = j + p
                wait(jj, p)
                off = (rbase + jj) * CH
                pltpu.sync_copy(buf_r[p], out_r_hbm.at[pl.ds(off, CH)])
                pltpu.sync_copy(buf_i[p], out_i_hbm.at[pl.ds(off, CH)])

                @pl.when(jj + 2 < nch)
                def _():
                    start(jj + 2, p)

    return grab


def kernel(x, real_embed, imag_embed):
    batch, hist = x.shape
    feat = real_embed.shape[1]
    b_flat = batch * hist
    x2 = x.reshape(b_flat // CH, CH)
    real_bf = real_embed.astype(jnp.bfloat16)
    imag_bf = imag_embed.astype(jnp.bfloat16)
    fn = _build(b_flat // CH, feat)
    out_r, out_i = fn(x2, real_bf, imag_bf)
    return (out_r.reshape(batch, hist, feat),
            out_i.reshape(batch, hist, feat))
